# Initial kernel scaffold; baseline (speedup 1.0000x reference)
import jax
import jax.numpy as jnp
from jax.experimental import pallas as pl


def kernel(Target, Type, Location, T_Target, T_Type, T_Location, W0, b0, W1, b1, W2, b2):
    raise NotImplementedError("write your pallas kernel here")


# speedup vs baseline: 3.7554x; 3.7554x over previous
import functools

import jax
import jax.numpy as jnp
from jax import lax
from jax.experimental import pallas as pl
from jax.experimental.pallas import tpu as pltpu
from jax.experimental.pallas import tpu_sc as plsc

B = 16384
V = 100000
D = 128

_NC = 2
_NS = 16
_NW = _NC * _NS
_BPW = B // _NW


def _sc_gather3(t0, t1, t2, i0, i1, i2):
    mesh = plsc.VectorSubcoreMesh(
        core_axis_name="c", subcore_axis_name="s",
        num_cores=_NC, num_subcores=_NS)

    @functools.partial(
        pl.kernel,
        out_type=(
            jax.ShapeDtypeStruct((B, D), jnp.float32),
            jax.ShapeDtypeStruct((B, D), jnp.float32),
            jax.ShapeDtypeStruct((B, D), jnp.float32),
        ),
        mesh=mesh,
        scratch_types=[
            pltpu.VMEM((_BPW,), jnp.int32),
            pltpu.VMEM((_BPW, D), jnp.float32),
            pltpu.SemaphoreType.DMA,
        ],
    )
    def gather_kernel(t0_h, t1_h, t2_h, i0_h, i1_h, i2_h,
                      o0_h, o1_h, o2_h, idx_v, rows_v, sem):
        wid = lax.axis_index("s") * _NC + lax.axis_index("c")
        base = wid * _BPW
        for tab, idx, out in ((t0_h, i0_h, o0_h),
                              (t1_h, i1_h, o1_h),
                              (t2_h, i2_h, o2_h)):
            pltpu.sync_copy(idx.at[pl.ds(base, _BPW)], idx_v)
            pltpu.async_copy(tab.at[idx_v], rows_v, sem).wait()
            pltpu.sync_copy(rows_v, out.at[pl.ds(base, _BPW)])

    return gather_kernel(t0, t1, t2, i0, i1, i2)


_BS = 1024


def _tc_body(e0_r, e1_r, e2_r, w0_r, w1_r, w2_r, b_r, out_r):
    acc = jnp.dot(e0_r[...], w0_r[...], preferred_element_type=jnp.float32)
    acc += jnp.dot(e1_r[...], w1_r[...], preferred_element_type=jnp.float32)
    acc += jnp.dot(e2_r[...], w2_r[...], preferred_element_type=jnp.float32)
    out_r[...] = acc + b_r[...]


def _tc_matmul(e0, e1, e2, w0, w1, w2, bsum):
    eb = pl.BlockSpec((_BS, D), lambda i: (i, 0))
    wb = pl.BlockSpec((D, D), lambda i: (0, 0))
    bb = pl.BlockSpec((1, D), lambda i: (0, 0))
    return pl.pallas_call(
        _tc_body,
        grid=(B // _BS,),
        in_specs=[eb, eb, eb, wb, wb, wb, bb],
        out_specs=pl.BlockSpec((_BS, D), lambda i: (i, 0)),
        out_shape=jax.ShapeDtypeStruct((B, D), jnp.float32),
        compiler_params=pltpu.CompilerParams(
            dimension_semantics=("arbitrary",)),
    )(e0, e1, e2, w0, w1, w2, bsum)


def kernel(Target, Type, Location, T_Target, T_Type, T_Location,
           W0, b0, W1, b1, W2, b2):
    i0 = Target.astype(jnp.int32)
    i1 = Type.astype(jnp.int32)
    i2 = Location.astype(jnp.int32)
    e0, e1, e2 = _sc_gather3(T_Target, T_Type, T_Location, i0, i1, i2)
    bsum = (b0 + b1 + b2).reshape(1, D)
    return _tc_matmul(e0, e1, e2, W0, W1, W2, bsum)


# speedup vs baseline: 4.0659x; 1.0827x over previous
import functools

import jax
import jax.numpy as jnp
from jax import lax
from jax.experimental import pallas as pl
from jax.experimental.pallas import tpu as pltpu
from jax.experimental.pallas import tpu_sc as plsc

B = 16384
V = 100000
D = 128

_NC = 2
_NS = 16
_NW = _NC * _NS
_BPW = B // _NW


def _sc_gather3(t0, t1, t2, i0, i1, i2):
    mesh = plsc.VectorSubcoreMesh(
        core_axis_name="c", subcore_axis_name="s",
        num_cores=_NC, num_subcores=_NS)

    @functools.partial(
        pl.kernel,
        out_type=(
            jax.ShapeDtypeStruct((B, D), jnp.float32),
            jax.ShapeDtypeStruct((B, D), jnp.float32),
            jax.ShapeDtypeStruct((B, D), jnp.float32),
        ),
        mesh=mesh,
        scratch_types=[
            pltpu.VMEM((_BPW,), jnp.int32),
            pltpu.VMEM((_BPW, D), jnp.float32),
            pltpu.SemaphoreType.DMA,
        ],
    )
    def gather_kernel(t0_h, t1_h, t2_h, i0_h, i1_h, i2_h,
                      o0_h, o1_h, o2_h, idx_v, rows_v, sem):
        wid = lax.axis_index("s") * _NC + lax.axis_index("c")
        base = wid * _BPW
        for tab, idx, out in ((t0_h, i0_h, o0_h),
                              (t1_h, i1_h, o1_h),
                              (t2_h, i2_h, o2_h)):
            pltpu.sync_copy(idx.at[pl.ds(base, _BPW)], idx_v)
            pltpu.async_copy(tab.at[idx_v], rows_v, sem).wait()
            pltpu.sync_copy(rows_v, out.at[pl.ds(base, _BPW)])

    return gather_kernel(t0, t1, t2, i0, i1, i2)


_BS = 2048


def _tc_body(e0_r, e1_r, e2_r, w0_r, w1_r, w2_r, b_r, out_r):
    acc = jnp.dot(e0_r[...], w0_r[...], preferred_element_type=jnp.float32)
    acc += jnp.dot(e1_r[...], w1_r[...], preferred_element_type=jnp.float32)
    acc += jnp.dot(e2_r[...], w2_r[...], preferred_element_type=jnp.float32)
    out_r[...] = acc + b_r[...]


def _tc_matmul(e0, e1, e2, w0, w1, w2, bsum):
    eb = pl.BlockSpec((_BS, D), lambda i: (i, 0))
    wb = pl.BlockSpec((D, D), lambda i: (0, 0))
    bb = pl.BlockSpec((1, D), lambda i: (0, 0))
    return pl.pallas_call(
        _tc_body,
        grid=(B // _BS,),
        in_specs=[eb, eb, eb, wb, wb, wb, bb],
        out_specs=pl.BlockSpec((_BS, D), lambda i: (i, 0)),
        out_shape=jax.ShapeDtypeStruct((B, D), jnp.float32),
        compiler_params=pltpu.CompilerParams(
            dimension_semantics=("arbitrary",)),
    )(e0, e1, e2, w0, w1, w2, bsum)


def kernel(Target, Type, Location, T_Target, T_Type, T_Location,
           W0, b0, W1, b1, W2, b2):
    i0 = Target.astype(jnp.int32)
    i1 = Type.astype(jnp.int32)
    i2 = Location.astype(jnp.int32)
    e0, e1, e2 = _sc_gather3(T_Target, T_Type, T_Location, i0, i1, i2)
    bsum = (b0 + b1 + b2).reshape(1, D)
    return _tc_matmul(e0, e1, e2, W0, W1, W2, bsum)


# speedup vs baseline: 4.1873x; 1.0299x over previous
import functools

import jax
import jax.numpy as jnp
from jax import lax
from jax.experimental import pallas as pl
from jax.experimental.pallas import tpu as pltpu
from jax.experimental.pallas import tpu_sc as plsc

B = 16384
V = 100000
D = 128

_NC = 2
_NS = 16
_NW = _NC * _NS
_BPW = B // _NW


def _sc_gather3(t0, t1, t2, i0, i1, i2):
    mesh = plsc.VectorSubcoreMesh(
        core_axis_name="c", subcore_axis_name="s",
        num_cores=_NC, num_subcores=_NS)

    @functools.partial(
        pl.kernel,
        out_type=(
            jax.ShapeDtypeStruct((B, D), jnp.float32),
            jax.ShapeDtypeStruct((B, D), jnp.float32),
            jax.ShapeDtypeStruct((B, D), jnp.float32),
        ),
        mesh=mesh,
        scratch_types=[
            pltpu.VMEM((_BPW,), jnp.int32),
            pltpu.VMEM((_BPW, D), jnp.float32),
            pltpu.SemaphoreType.DMA,
        ],
    )
    def gather_kernel(t0_h, t1_h, t2_h, i0_h, i1_h, i2_h,
                      o0_h, o1_h, o2_h, idx_v, rows_v, sem):
        wid = lax.axis_index("s") * _NC + lax.axis_index("c")
        base = wid * _BPW
        for tab, idx, out in ((t0_h, i0_h, o0_h),
                              (t1_h, i1_h, o1_h),
                              (t2_h, i2_h, o2_h)):
            pltpu.sync_copy(idx.at[pl.ds(base, _BPW)], idx_v)
            pltpu.async_copy(tab.at[idx_v], rows_v, sem).wait()
            pltpu.sync_copy(rows_v, out.at[pl.ds(base, _BPW)])

    return gather_kernel(t0, t1, t2, i0, i1, i2)


_BS = 4096


def _tc_body(e0_r, e1_r, e2_r, w0_r, w1_r, w2_r, b_r, out_r):
    acc = jnp.dot(e0_r[...], w0_r[...], preferred_element_type=jnp.float32)
    acc += jnp.dot(e1_r[...], w1_r[...], preferred_element_type=jnp.float32)
    acc += jnp.dot(e2_r[...], w2_r[...], preferred_element_type=jnp.float32)
    out_r[...] = acc + b_r[...]


def _tc_matmul(e0, e1, e2, w0, w1, w2, bsum):
    eb = pl.BlockSpec((_BS, D), lambda i: (i, 0))
    wb = pl.BlockSpec((D, D), lambda i: (0, 0))
    bb = pl.BlockSpec((1, D), lambda i: (0, 0))
    return pl.pallas_call(
        _tc_body,
        grid=(B // _BS,),
        in_specs=[eb, eb, eb, wb, wb, wb, bb],
        out_specs=pl.BlockSpec((_BS, D), lambda i: (i, 0)),
        out_shape=jax.ShapeDtypeStruct((B, D), jnp.float32),
        compiler_params=pltpu.CompilerParams(
            dimension_semantics=("arbitrary",)),
    )(e0, e1, e2, w0, w1, w2, bsum)


def kernel(Target, Type, Location, T_Target, T_Type, T_Location,
           W0, b0, W1, b1, W2, b2):
    i0 = Target.astype(jnp.int32)
    i1 = Type.astype(jnp.int32)
    i2 = Location.astype(jnp.int32)
    e0, e1, e2 = _sc_gather3(T_Target, T_Type, T_Location, i0, i1, i2)
    bsum = (b0 + b1 + b2).reshape(1, D)
    return _tc_matmul(e0, e1, e2, W0, W1, W2, bsum)


# speedup vs baseline: 4.2523x; 1.0155x over previous
import functools

import jax
import jax.numpy as jnp
from jax import lax
from jax.experimental import pallas as pl
from jax.experimental.pallas import tpu as pltpu
from jax.experimental.pallas import tpu_sc as plsc

B = 16384
V = 100000
D = 128

_NC = 2
_NS = 16
_NW = _NC * _NS
_BPW = B // _NW


def _sc_gather3(t0, t1, t2, i0, i1, i2):
    mesh = plsc.VectorSubcoreMesh(
        core_axis_name="c", subcore_axis_name="s",
        num_cores=_NC, num_subcores=_NS)

    @functools.partial(
        pl.kernel,
        out_type=(
            jax.ShapeDtypeStruct((B, D), jnp.float32),
            jax.ShapeDtypeStruct((B, D), jnp.float32),
            jax.ShapeDtypeStruct((B, D), jnp.float32),
        ),
        mesh=mesh,
        scratch_types=[
            pltpu.VMEM((_BPW,), jnp.int32),
            pltpu.VMEM((_BPW, D), jnp.float32),
            pltpu.SemaphoreType.DMA,
        ],
    )
    def gather_kernel(t0_h, t1_h, t2_h, i0_h, i1_h, i2_h,
                      o0_h, o1_h, o2_h, idx_v, rows_v, sem):
        wid = lax.axis_index("s") * _NC + lax.axis_index("c")
        base = wid * _BPW
        for tab, idx, out in ((t0_h, i0_h, o0_h),
                              (t1_h, i1_h, o1_h),
                              (t2_h, i2_h, o2_h)):
            pltpu.sync_copy(idx.at[pl.ds(base, _BPW)], idx_v)
            pltpu.async_copy(tab.at[idx_v], rows_v, sem).wait()
            pltpu.sync_copy(rows_v, out.at[pl.ds(base, _BPW)])

    return gather_kernel(t0, t1, t2, i0, i1, i2)


_BS = 8192


def _tc_body(e0_r, e1_r, e2_r, w0_r, w1_r, w2_r, b_r, out_r):
    acc = jnp.dot(e0_r[...], w0_r[...], preferred_element_type=jnp.float32)
    acc += jnp.dot(e1_r[...], w1_r[...], preferred_element_type=jnp.float32)
    acc += jnp.dot(e2_r[...], w2_r[...], preferred_element_type=jnp.float32)
    out_r[...] = acc + b_r[...]


def _tc_matmul(e0, e1, e2, w0, w1, w2, bsum):
    eb = pl.BlockSpec((_BS, D), lambda i: (i, 0))
    wb = pl.BlockSpec((D, D), lambda i: (0, 0))
    bb = pl.BlockSpec((1, D), lambda i: (0, 0))
    return pl.pallas_call(
        _tc_body,
        grid=(B // _BS,),
        in_specs=[eb, eb, eb, wb, wb, wb, bb],
        out_specs=pl.BlockSpec((_BS, D), lambda i: (i, 0)),
        out_shape=jax.ShapeDtypeStruct((B, D), jnp.float32),
        compiler_params=pltpu.CompilerParams(
            dimension_semantics=("arbitrary",)),
    )(e0, e1, e2, w0, w1, w2, bsum)


def kernel(Target, Type, Location, T_Target, T_Type, T_Location,
           W0, b0, W1, b1, W2, b2):
    i0 = Target.astype(jnp.int32)
    i1 = Type.astype(jnp.int32)
    i2 = Location.astype(jnp.int32)
    e0, e1, e2 = _sc_gather3(T_Target, T_Type, T_Location, i0, i1, i2)
    bsum = (b0 + b1 + b2).reshape(1, D)
    return _tc_matmul(e0, e1, e2, W0, W1, W2, bsum)
